# speedup vs baseline: 1.6650x; 1.6650x over previous
import functools

import jax
import jax.numpy as jnp
from jax import lax
from jax.experimental import pallas as pl
from jax.experimental.pallas import tpu as pltpu
from jax.experimental.pallas import tpu_sc as plsc

_N_NODES = 10000
_N_EDGES = 160000
_D_FEAT = 256
_D_EDGE = 16
_D_OUT = 256

_NC = 2
_NS = 16
_L = 16
_NW = _NC * _NS
_EPW = _N_EDGES // _NW
_C = 40
_NCHUNK = _EPW // _C


def _node_proj_body(nodes_ref, wr_ref, ws_ref, pr_ref, ps_ref):
    x = nodes_ref[...]
    pr_ref[...] = jnp.dot(x, wr_ref[...], preferred_element_type=jnp.float32)
    ps_ref[...] = jnp.dot(x, ws_ref[...], preferred_element_type=jnp.float32)


def _edge_proj_body(edges_ref, we_ref, b_ref, out_ref):
    out_ref[...] = (
        jnp.dot(edges_ref[...], we_ref[...], preferred_element_type=jnp.float32)
        + b_ref[...]
    )


def _node_projections(nodes, w_r, w_s):
    blk = 1000
    grid = _N_NODES // blk
    return pl.pallas_call(
        _node_proj_body,
        grid=(grid,),
        in_specs=[
            pl.BlockSpec((blk, _D_FEAT), lambda i: (i, 0)),
            pl.BlockSpec((_D_FEAT, _D_OUT), lambda i: (0, 0)),
            pl.BlockSpec((_D_FEAT, _D_OUT), lambda i: (0, 0)),
        ],
        out_specs=[
            pl.BlockSpec((blk, _D_OUT), lambda i: (i, 0)),
            pl.BlockSpec((blk, _D_OUT), lambda i: (i, 0)),
        ],
        out_shape=[
            jax.ShapeDtypeStruct((_N_NODES, _D_OUT), jnp.float32),
            jax.ShapeDtypeStruct((_N_NODES, _D_OUT), jnp.float32),
        ],
    )(nodes, w_r, w_s)


def _edge_projection(edges, w_e, b2d):
    blk = 8000
    grid = _N_EDGES // blk
    return pl.pallas_call(
        _edge_proj_body,
        grid=(grid,),
        in_specs=[
            pl.BlockSpec((blk, _D_EDGE), lambda i: (i, 0)),
            pl.BlockSpec((_D_EDGE, _D_OUT), lambda i: (0, 0)),
            pl.BlockSpec((1, _D_OUT), lambda i: (0, 0)),
        ],
        out_specs=pl.BlockSpec((blk, _D_OUT), lambda i: (i, 0)),
        out_shape=jax.ShapeDtypeStruct((_N_EDGES, _D_OUT), jnp.float32),
    )(edges, w_e, b2d)


def _sc_body(pr_hbm, ps_hbm, eb_hbm, recv_hbm, send_hbm, out_hbm,
             idx_r, idx_s, r_v, s_v, e_v, sem):
    wid = lax.axis_index("s") * _NC + lax.axis_index("c")
    base = wid * _EPW
    pltpu.sync_copy(recv_hbm.at[pl.ds(base, _EPW)], idx_r)
    pltpu.sync_copy(send_hbm.at[pl.ds(base, _EPW)], idx_s)

    def chunk_body(ci, carry):
        off = base + ci * _C
        cp_r = pltpu.async_copy(pr_hbm.at[idx_r.at[pl.ds(ci * _C, _C)]], r_v, sem)
        cp_s = pltpu.async_copy(ps_hbm.at[idx_s.at[pl.ds(ci * _C, _C)]], s_v, sem)
        cp_e = pltpu.async_copy(eb_hbm.at[pl.ds(off, _C)], e_v, sem)
        cp_r.wait()
        cp_s.wait()
        cp_e.wait()

        def row_body(i, c2):
            for j in range(_D_OUT // _L):
                sl = pl.ds(j * _L, _L)
                e_v[i, sl] = e_v[i, sl] + r_v[i, sl] + s_v[i, sl]
            return c2

        lax.fori_loop(0, _C, row_body, 0)
        pltpu.sync_copy(e_v, out_hbm.at[pl.ds(off, _C)])
        return carry

    lax.fori_loop(0, _NCHUNK, chunk_body, 0)


_sc_gather_add = functools.partial(
    pl.kernel,
    out_type=jax.ShapeDtypeStruct((_N_EDGES, _D_OUT), jnp.float32),
    mesh=plsc.VectorSubcoreMesh(core_axis_name="c", subcore_axis_name="s"),
    scratch_types=[
        pltpu.VMEM((_EPW,), jnp.int32),
        pltpu.VMEM((_EPW,), jnp.int32),
        pltpu.VMEM((_C, _D_OUT), jnp.float32),
        pltpu.VMEM((_C, _D_OUT), jnp.float32),
        pltpu.VMEM((_C, _D_OUT), jnp.float32),
        pltpu.SemaphoreType.DMA,
    ],
)(_sc_body)


def kernel(edges, nodes, W, b, senders, receivers):
    w_e = W[:_D_EDGE]
    w_r = W[_D_EDGE:_D_EDGE + _D_FEAT]
    w_s = W[_D_EDGE + _D_FEAT:]
    pr, ps = _node_projections(nodes, w_r, w_s)
    eb = _edge_projection(edges, w_e, b.reshape(1, _D_OUT))
    return _sc_gather_add(pr, ps, eb, receivers, senders)


# speedup vs baseline: 2.5640x; 1.5399x over previous
import functools

import jax
import jax.numpy as jnp
from jax import lax
from jax.experimental import pallas as pl
from jax.experimental.pallas import tpu as pltpu
from jax.experimental.pallas import tpu_sc as plsc

_N_NODES = 10000
_N_EDGES = 160000
_D_FEAT = 256
_D_EDGE = 16
_D_OUT = 256

_NC = 2
_NS = 16
_L = 16
_NW = _NC * _NS
_EPW = _N_EDGES // _NW
_C = 40
_NCHUNK = _EPW // _C


def _node_proj_body(nodes_ref, wr_ref, ws_ref, pr_ref, ps_ref):
    x = nodes_ref[...]
    pr_ref[...] = jnp.dot(x, wr_ref[...], preferred_element_type=jnp.float32)
    ps_ref[...] = jnp.dot(x, ws_ref[...], preferred_element_type=jnp.float32)


def _edge_proj_body(edges_ref, we_ref, b_ref, out_ref):
    out_ref[...] = (
        jnp.dot(edges_ref[...], we_ref[...], preferred_element_type=jnp.float32)
        + b_ref[...]
    )


def _node_projections(nodes, w_r, w_s):
    blk = 1000
    grid = _N_NODES // blk
    return pl.pallas_call(
        _node_proj_body,
        grid=(grid,),
        in_specs=[
            pl.BlockSpec((blk, _D_FEAT), lambda i: (i, 0)),
            pl.BlockSpec((_D_FEAT, _D_OUT), lambda i: (0, 0)),
            pl.BlockSpec((_D_FEAT, _D_OUT), lambda i: (0, 0)),
        ],
        out_specs=[
            pl.BlockSpec((blk, _D_OUT), lambda i: (i, 0)),
            pl.BlockSpec((blk, _D_OUT), lambda i: (i, 0)),
        ],
        out_shape=[
            jax.ShapeDtypeStruct((_N_NODES, _D_OUT), jnp.float32),
            jax.ShapeDtypeStruct((_N_NODES, _D_OUT), jnp.float32),
        ],
    )(nodes, w_r, w_s)


def _edge_projection(edges, w_e, b2d):
    blk = 8000
    grid = _N_EDGES // blk
    return pl.pallas_call(
        _edge_proj_body,
        grid=(grid,),
        in_specs=[
            pl.BlockSpec((blk, _D_EDGE), lambda i: (i, 0)),
            pl.BlockSpec((_D_EDGE, _D_OUT), lambda i: (0, 0)),
            pl.BlockSpec((1, _D_OUT), lambda i: (0, 0)),
        ],
        out_specs=pl.BlockSpec((blk, _D_OUT), lambda i: (i, 0)),
        out_shape=jax.ShapeDtypeStruct((_N_EDGES, _D_OUT), jnp.float32),
    )(edges, w_e, b2d)


def _sc_body(pr_hbm, ps_hbm, eb_hbm, recv_hbm, send_hbm, out_hbm,
             idx_r, idx_s,
             r0, s0, e0, o0, r1, s1, e1, o1,
             gsem0, ssem0, gsem1, ssem1):
    wid = lax.axis_index("s") * _NC + lax.axis_index("c")
    base = wid * _EPW
    pltpu.sync_copy(recv_hbm.at[pl.ds(base, _EPW)], idx_r)
    pltpu.sync_copy(send_hbm.at[pl.ds(base, _EPW)], idx_s)

    bufs = ((r0, s0, e0, o0, gsem0, ssem0), (r1, s1, e1, o1, gsem1, ssem1))

    def start_gather(ci, b):
        r, s, e, _o, gs, _ss = bufs[b]
        pltpu.async_copy(pr_hbm.at[idx_r.at[pl.ds(ci * _C, _C)]], r, gs)
        pltpu.async_copy(ps_hbm.at[idx_s.at[pl.ds(ci * _C, _C)]], s, gs)
        pltpu.async_copy(eb_hbm.at[pl.ds(base + ci * _C, _C)], e, gs)

    def wait_gather(b):
        r, s, e, _o, gs, _ss = bufs[b]
        pltpu.make_async_copy(pr_hbm.at[pl.ds(0, _C)], r, gs).wait()
        pltpu.make_async_copy(ps_hbm.at[pl.ds(0, _C)], s, gs).wait()
        pltpu.make_async_copy(eb_hbm.at[pl.ds(0, _C)], e, gs).wait()

    def start_store(ci, b):
        _r, _s, _e, o, _gs, ss = bufs[b]
        pltpu.async_copy(o, out_hbm.at[pl.ds(base + ci * _C, _C)], ss)

    def wait_store(b):
        _r, _s, _e, o, _gs, ss = bufs[b]
        pltpu.make_async_copy(o, out_hbm.at[pl.ds(0, _C)], ss).wait()

    def add_chunk(b):
        r, s, e, o, _gs, _ss = bufs[b]

        @plsc.parallel_loop(0, _C, 1, unroll=2)
        def _row(i):
            for j in range(_D_OUT // _L):
                sl = pl.ds(j * _L, _L)
                o[i, sl] = e[i, sl] + r[i, sl] + s[i, sl]

    start_gather(0, 0)
    start_gather(1, 1)

    def pair(g, carry):
        for b in (0, 1):
            ci = 2 * g + b
            wait_gather(b)

            @pl.when(g > 0)
            def _():
                wait_store(b)

            add_chunk(b)

            @pl.when(ci + 2 < _NCHUNK)
            def _():
                start_gather(ci + 2, b)

            start_store(ci, b)
        return carry

    lax.fori_loop(0, _NCHUNK // 2, pair, 0)

    wait_gather(0)
    wait_store(0)
    add_chunk(0)
    start_store(_NCHUNK - 1, 0)
    wait_store(0)
    wait_store(1)


_sc_gather_add = functools.partial(
    pl.kernel,
    out_type=jax.ShapeDtypeStruct((_N_EDGES, _D_OUT), jnp.float32),
    mesh=plsc.VectorSubcoreMesh(core_axis_name="c", subcore_axis_name="s"),
    scratch_types=[
        pltpu.VMEM((_EPW,), jnp.int32),
        pltpu.VMEM((_EPW,), jnp.int32),
        pltpu.VMEM((_C, _D_OUT), jnp.float32),
        pltpu.VMEM((_C, _D_OUT), jnp.float32),
        pltpu.VMEM((_C, _D_OUT), jnp.float32),
        pltpu.VMEM((_C, _D_OUT), jnp.float32),
        pltpu.VMEM((_C, _D_OUT), jnp.float32),
        pltpu.VMEM((_C, _D_OUT), jnp.float32),
        pltpu.VMEM((_C, _D_OUT), jnp.float32),
        pltpu.VMEM((_C, _D_OUT), jnp.float32),
        pltpu.SemaphoreType.DMA,
        pltpu.SemaphoreType.DMA,
        pltpu.SemaphoreType.DMA,
        pltpu.SemaphoreType.DMA,
    ],
)(_sc_body)


def kernel(edges, nodes, W, b, senders, receivers):
    w_e = W[:_D_EDGE]
    w_r = W[_D_EDGE:_D_EDGE + _D_FEAT]
    w_s = W[_D_EDGE + _D_FEAT:]
    pr, ps = _node_projections(nodes, w_r, w_s)
    eb = _edge_projection(edges, w_e, b.reshape(1, _D_OUT))
    return _sc_gather_add(pr, ps, eb, receivers, senders)


# speedup vs baseline: 3.4050x; 1.3280x over previous
import functools

import jax
import jax.numpy as jnp
from jax import lax
from jax.experimental import pallas as pl
from jax.experimental.pallas import tpu as pltpu
from jax.experimental.pallas import tpu_sc as plsc

_N_NODES = 10000
_N_EDGES = 160000
_D_FEAT = 256
_D_EDGE = 16
_D_OUT = 256

_NC = 2
_NS = 16
_L = 16
_NW = _NC * _NS
_EPW = _N_EDGES // _NW
_C = 40
_NCHUNK = _EPW // _C


def _pack_pairs(x):
    u = jax.lax.bitcast_convert_type(x.astype(jnp.bfloat16), jnp.uint16)
    lo = u[:, :_D_OUT // 2].astype(jnp.uint32)
    hi = u[:, _D_OUT // 2:].astype(jnp.uint32)
    return jax.lax.bitcast_convert_type(lo | (hi << 16), jnp.int32)


def _node_proj_body(nodes_ref, wr_ref, ws_ref, pr_ref, ps_ref):
    x = nodes_ref[...]
    pr_ref[...] = _pack_pairs(
        jnp.dot(x, wr_ref[...], preferred_element_type=jnp.float32))
    ps_ref[...] = _pack_pairs(
        jnp.dot(x, ws_ref[...], preferred_element_type=jnp.float32))


def _edge_proj_body(edges_ref, we_ref, b_ref, out_ref):
    out_ref[...] = _pack_pairs(
        jnp.dot(edges_ref[...], we_ref[...], preferred_element_type=jnp.float32)
        + b_ref[...])


def _node_projections(nodes, w_r, w_s):
    blk = 1000
    grid = _N_NODES // blk
    return pl.pallas_call(
        _node_proj_body,
        grid=(grid,),
        in_specs=[
            pl.BlockSpec((blk, _D_FEAT), lambda i: (i, 0)),
            pl.BlockSpec((_D_FEAT, _D_OUT), lambda i: (0, 0)),
            pl.BlockSpec((_D_FEAT, _D_OUT), lambda i: (0, 0)),
        ],
        out_specs=[
            pl.BlockSpec((blk, _D_OUT // 2), lambda i: (i, 0)),
            pl.BlockSpec((blk, _D_OUT // 2), lambda i: (i, 0)),
        ],
        out_shape=[
            jax.ShapeDtypeStruct((_N_NODES, _D_OUT // 2), jnp.int32),
            jax.ShapeDtypeStruct((_N_NODES, _D_OUT // 2), jnp.int32),
        ],
    )(nodes, w_r, w_s)


def _edge_projection(edges, w_e, b2d):
    blk = 8000
    grid = _N_EDGES // blk
    return pl.pallas_call(
        _edge_proj_body,
        grid=(grid,),
        in_specs=[
            pl.BlockSpec((blk, _D_EDGE), lambda i: (i, 0)),
            pl.BlockSpec((_D_EDGE, _D_OUT), lambda i: (0, 0)),
            pl.BlockSpec((1, _D_OUT), lambda i: (0, 0)),
        ],
        out_specs=pl.BlockSpec((blk, _D_OUT // 2), lambda i: (i, 0)),
        out_shape=jax.ShapeDtypeStruct((_N_EDGES, _D_OUT // 2), jnp.int32),
    )(edges, w_e, b2d)


def _sc_body(pr_hbm, ps_hbm, eb_hbm, recv_hbm, send_hbm, out_hbm,
             idx_r, idx_s,
             r0, s0, e0, o0, r1, s1, e1, o1,
             gsem0, ssem0, gsem1, ssem1):
    wid = lax.axis_index("s") * _NC + lax.axis_index("c")
    base = wid * _EPW
    pltpu.sync_copy(recv_hbm.at[pl.ds(base, _EPW)], idx_r)
    pltpu.sync_copy(send_hbm.at[pl.ds(base, _EPW)], idx_s)

    bufs = ((r0, s0, e0, o0, gsem0, ssem0), (r1, s1, e1, o1, gsem1, ssem1))

    def start_gather(ci, b):
        r, s, e, _o, gs, _ss = bufs[b]
        pltpu.async_copy(pr_hbm.at[idx_r.at[pl.ds(ci * _C, _C)]], r, gs)
        pltpu.async_copy(ps_hbm.at[idx_s.at[pl.ds(ci * _C, _C)]], s, gs)
        pltpu.async_copy(eb_hbm.at[pl.ds(base + ci * _C, _C)], e, gs)

    def wait_gather(b):
        r, s, e, _o, gs, _ss = bufs[b]
        pltpu.make_async_copy(pr_hbm.at[pl.ds(0, _C)], r, gs).wait()
        pltpu.make_async_copy(ps_hbm.at[pl.ds(0, _C)], s, gs).wait()
        pltpu.make_async_copy(eb_hbm.at[pl.ds(0, _C)], e, gs).wait()

    def start_store(ci, b):
        _r, _s, _e, o, _gs, ss = bufs[b]
        pltpu.async_copy(o, out_hbm.at[pl.ds(base + ci * _C, _C)], ss)

    def wait_store(b):
        _r, _s, _e, o, _gs, ss = bufs[b]
        pltpu.make_async_copy(o, out_hbm.at[pl.ds(0, _C)], ss).wait()

    def add_chunk(b):
        r, s, e, o, _gs, _ss = bufs[b]

        hi_mask = jnp.int32(-65536)

        def halves(w):
            lo = jax.lax.bitcast_convert_type(w << 16, jnp.float32)
            hi = jax.lax.bitcast_convert_type(w & hi_mask, jnp.float32)
            return lo, hi

        @plsc.parallel_loop(0, _C, 1, unroll=2)
        def _row(i):
            for j in range(_D_OUT // (2 * _L)):
                sl = pl.ds(j * _L, _L)
                ra, rb = halves(r[i, sl])
                sa, sb = halves(s[i, sl])
                ea, ebv = halves(e[i, sl])
                o[i, pl.ds(j * _L, _L)] = ea + ra + sa
                o[i, pl.ds(_D_OUT // 2 + j * _L, _L)] = ebv + rb + sb

    start_gather(0, 0)
    start_gather(1, 1)

    def pair(g, carry):
        for b in (0, 1):
            ci = 2 * g + b
            wait_gather(b)

            @pl.when(g > 0)
            def _():
                wait_store(b)

            add_chunk(b)

            @pl.when(ci + 2 < _NCHUNK)
            def _():
                start_gather(ci + 2, b)

            start_store(ci, b)
        return carry

    lax.fori_loop(0, _NCHUNK // 2, pair, 0)

    wait_gather(0)
    wait_store(0)
    add_chunk(0)
    start_store(_NCHUNK - 1, 0)
    wait_store(0)
    wait_store(1)


_sc_gather_add = functools.partial(
    pl.kernel,
    out_type=jax.ShapeDtypeStruct((_N_EDGES, _D_OUT), jnp.float32),
    mesh=plsc.VectorSubcoreMesh(core_axis_name="c", subcore_axis_name="s"),
    scratch_types=[
        pltpu.VMEM((_EPW,), jnp.int32),
        pltpu.VMEM((_EPW,), jnp.int32),
        pltpu.VMEM((_C, _D_OUT // 2), jnp.int32),
        pltpu.VMEM((_C, _D_OUT // 2), jnp.int32),
        pltpu.VMEM((_C, _D_OUT // 2), jnp.int32),
        pltpu.VMEM((_C, _D_OUT), jnp.float32),
        pltpu.VMEM((_C, _D_OUT // 2), jnp.int32),
        pltpu.VMEM((_C, _D_OUT // 2), jnp.int32),
        pltpu.VMEM((_C, _D_OUT // 2), jnp.int32),
        pltpu.VMEM((_C, _D_OUT), jnp.float32),
        pltpu.SemaphoreType.DMA,
        pltpu.SemaphoreType.DMA,
        pltpu.SemaphoreType.DMA,
        pltpu.SemaphoreType.DMA,
    ],
)(_sc_body)


def kernel(edges, nodes, W, b, senders, receivers):
    w_e = W[:_D_EDGE]
    w_r = W[_D_EDGE:_D_EDGE + _D_FEAT]
    w_s = W[_D_EDGE + _D_FEAT:]
    pr, ps = _node_projections(nodes, w_r, w_s)
    eb = _edge_projection(edges, w_e, b.reshape(1, _D_OUT))
    return _sc_gather_add(pr, ps, eb, receivers, senders)


# speedup vs baseline: 3.4563x; 1.0151x over previous
import functools

import jax
import jax.numpy as jnp
from jax import lax
from jax.experimental import pallas as pl
from jax.experimental.pallas import tpu as pltpu
from jax.experimental.pallas import tpu_sc as plsc

_N_NODES = 10000
_N_EDGES = 160000
_D_FEAT = 256
_D_EDGE = 16
_D_OUT = 256

_NC = 2
_NS = 16
_L = 16
_NW = _NC * _NS
_EPW = _N_EDGES // _NW
_C = 40
_NCHUNK = _EPW // _C


def _pack_pairs(x):
    u = jax.lax.bitcast_convert_type(x.astype(jnp.bfloat16), jnp.uint16)
    lo = u[:, :_D_OUT // 2].astype(jnp.uint32)
    hi = u[:, _D_OUT // 2:].astype(jnp.uint32)
    return jax.lax.bitcast_convert_type(lo | (hi << 16), jnp.int32)


def _node_proj_body(nodes_ref, w_ref, pr_ref, ps_ref):
    x = nodes_ref[...]
    w_r = w_ref[_D_EDGE:_D_EDGE + _D_FEAT, :]
    w_s = w_ref[_D_EDGE + _D_FEAT:, :]
    pr_ref[...] = _pack_pairs(
        jnp.dot(x, w_r, preferred_element_type=jnp.float32))
    ps_ref[...] = _pack_pairs(
        jnp.dot(x, w_s, preferred_element_type=jnp.float32))


def _edge_proj_body(edges_ref, w_ref, b_ref, out_ref):
    w_e = w_ref[:_D_EDGE, :]
    out_ref[...] = _pack_pairs(
        jnp.dot(edges_ref[...], w_e, preferred_element_type=jnp.float32)
        + b_ref[...])


def _node_projections(nodes, w):
    blk = 1000
    grid = _N_NODES // blk
    return pl.pallas_call(
        _node_proj_body,
        grid=(grid,),
        in_specs=[
            pl.BlockSpec((blk, _D_FEAT), lambda i: (i, 0)),
            pl.BlockSpec(w.shape, lambda i: (0, 0)),
        ],
        out_specs=[
            pl.BlockSpec((blk, _D_OUT // 2), lambda i: (i, 0)),
            pl.BlockSpec((blk, _D_OUT // 2), lambda i: (i, 0)),
        ],
        out_shape=[
            jax.ShapeDtypeStruct((_N_NODES, _D_OUT // 2), jnp.int32),
            jax.ShapeDtypeStruct((_N_NODES, _D_OUT // 2), jnp.int32),
        ],
    )(nodes, w)


def _edge_projection(edges, w, b2d):
    blk = 8000
    grid = _N_EDGES // blk
    return pl.pallas_call(
        _edge_proj_body,
        grid=(grid,),
        in_specs=[
            pl.BlockSpec((blk, _D_EDGE), lambda i: (i, 0)),
            pl.BlockSpec(w.shape, lambda i: (0, 0)),
            pl.BlockSpec((1, _D_OUT), lambda i: (0, 0)),
        ],
        out_specs=pl.BlockSpec((blk, _D_OUT // 2), lambda i: (i, 0)),
        out_shape=jax.ShapeDtypeStruct((_N_EDGES, _D_OUT // 2), jnp.int32),
    )(edges, w, b2d)


def _sc_body(pr_hbm, ps_hbm, eb_hbm, recv_hbm, send_hbm, out_hbm,
             idx_r, idx_s,
             r0, s0, e0, o0, r1, s1, e1, o1,
             gsem0, ssem0, gsem1, ssem1):
    wid = lax.axis_index("s") * _NC + lax.axis_index("c")
    base = wid * _EPW
    pltpu.sync_copy(recv_hbm.at[pl.ds(base, _EPW)], idx_r)
    pltpu.sync_copy(send_hbm.at[pl.ds(base, _EPW)], idx_s)

    bufs = ((r0, s0, e0, o0, gsem0, ssem0), (r1, s1, e1, o1, gsem1, ssem1))

    def start_gather(ci, b):
        r, s, e, _o, gs, _ss = bufs[b]
        pltpu.async_copy(pr_hbm.at[idx_r.at[pl.ds(ci * _C, _C)]], r, gs)
        pltpu.async_copy(ps_hbm.at[idx_s.at[pl.ds(ci * _C, _C)]], s, gs)
        pltpu.async_copy(eb_hbm.at[pl.ds(base + ci * _C, _C)], e, gs)

    def wait_gather(b):
        r, s, e, _o, gs, _ss = bufs[b]
        pltpu.make_async_copy(pr_hbm.at[pl.ds(0, _C)], r, gs).wait()
        pltpu.make_async_copy(ps_hbm.at[pl.ds(0, _C)], s, gs).wait()
        pltpu.make_async_copy(eb_hbm.at[pl.ds(0, _C)], e, gs).wait()

    def start_store(ci, b):
        _r, _s, _e, o, _gs, ss = bufs[b]
        pltpu.async_copy(o, out_hbm.at[pl.ds(base + ci * _C, _C)], ss)

    def wait_store(b):
        _r, _s, _e, o, _gs, ss = bufs[b]
        pltpu.make_async_copy(o, out_hbm.at[pl.ds(0, _C)], ss).wait()

    def add_chunk(b):
        r, s, e, o, _gs, _ss = bufs[b]

        def halves(w):
            lo = jax.lax.bitcast_convert_type(w << 16, jnp.float32)
            hi = jax.lax.bitcast_convert_type(w, jnp.float32)
            return lo, hi

        @plsc.parallel_loop(0, _C, 1, unroll=4)
        def _row(i):
            for j in range(_D_OUT // (2 * _L)):
                sl = pl.ds(j * _L, _L)
                ra, rb = halves(r[i, sl])
                sa, sb = halves(s[i, sl])
                ea, ebv = halves(e[i, sl])
                o[i, pl.ds(j * _L, _L)] = ea + ra + sa
                o[i, pl.ds(_D_OUT // 2 + j * _L, _L)] = ebv + rb + sb

    start_gather(0, 0)
    start_gather(1, 1)

    def pair(g, carry):
        for b in (0, 1):
            ci = 2 * g + b
            wait_gather(b)

            @pl.when(g > 0)
            def _():
                wait_store(b)

            add_chunk(b)

            @pl.when(ci + 2 < _NCHUNK)
            def _():
                start_gather(ci + 2, b)

            start_store(ci, b)
        return carry

    lax.fori_loop(0, _NCHUNK // 2, pair, 0)

    wait_gather(0)
    wait_store(0)
    add_chunk(0)
    start_store(_NCHUNK - 1, 0)
    wait_store(0)
    wait_store(1)


_sc_gather_add = functools.partial(
    pl.kernel,
    out_type=jax.ShapeDtypeStruct((_N_EDGES, _D_OUT), jnp.float32),
    mesh=plsc.VectorSubcoreMesh(core_axis_name="c", subcore_axis_name="s"),
    scratch_types=[
        pltpu.VMEM((_EPW,), jnp.int32),
        pltpu.VMEM((_EPW,), jnp.int32),
        pltpu.VMEM((_C, _D_OUT // 2), jnp.int32),
        pltpu.VMEM((_C, _D_OUT // 2), jnp.int32),
        pltpu.VMEM((_C, _D_OUT // 2), jnp.int32),
        pltpu.VMEM((_C, _D_OUT), jnp.float32),
        pltpu.VMEM((_C, _D_OUT // 2), jnp.int32),
        pltpu.VMEM((_C, _D_OUT // 2), jnp.int32),
        pltpu.VMEM((_C, _D_OUT // 2), jnp.int32),
        pltpu.VMEM((_C, _D_OUT), jnp.float32),
        pltpu.SemaphoreType.DMA,
        pltpu.SemaphoreType.DMA,
        pltpu.SemaphoreType.DMA,
        pltpu.SemaphoreType.DMA,
    ],
)(_sc_body)


def kernel(edges, nodes, W, b, senders, receivers):
    pr, ps = _node_projections(nodes, W)
    eb = _edge_projection(edges, W, b.reshape(1, _D_OUT))
    return _sc_gather_add(pr, ps, eb, receivers, senders)


# speedup vs baseline: 4.4292x; 1.2815x over previous
import functools

import jax
import jax.numpy as jnp
from jax import lax
from jax.experimental import pallas as pl
from jax.experimental.pallas import tpu as pltpu
from jax.experimental.pallas import tpu_sc as plsc

_N_NODES = 10000
_N_EDGES = 160000
_D_FEAT = 256
_D_EDGE = 16
_D_OUT = 256

_NC = 2
_NS = 16
_L = 16
_NW = _NC * _NS
_EPW = _N_EDGES // _NW
_C = 40
_NCHUNK = _EPW // _C


def _pack_pairs(x):
    u = jax.lax.bitcast_convert_type(x.astype(jnp.bfloat16), jnp.uint16)
    lo = u[:, :_D_OUT // 2].astype(jnp.uint32)
    hi = u[:, _D_OUT // 2:].astype(jnp.uint32)
    return jax.lax.bitcast_convert_type(lo | (hi << 16), jnp.int32)


def _node_proj_body(nodes_ref, w_ref, pr_ref, ps_ref):
    x = nodes_ref[...]
    w_r = w_ref[_D_EDGE:_D_EDGE + _D_FEAT, :]
    w_s = w_ref[_D_EDGE + _D_FEAT:, :]
    pr_ref[...] = _pack_pairs(
        jnp.dot(x, w_r, preferred_element_type=jnp.float32))
    ps_ref[...] = _pack_pairs(
        jnp.dot(x, w_s, preferred_element_type=jnp.float32))


def _edge_proj_body(edges_t_ref, w_ref, b_ref, out_ref):
    w_e = w_ref[:_D_EDGE, :]
    prod = jax.lax.dot_general(
        edges_t_ref[...], w_e, (((0,), (0,)), ((), ())),
        preferred_element_type=jnp.float32)
    out_ref[...] = _pack_pairs(prod + b_ref[...])


def _node_projections(nodes, w):
    blk = 1000
    grid = _N_NODES // blk
    return pl.pallas_call(
        _node_proj_body,
        grid=(grid,),
        in_specs=[
            pl.BlockSpec((blk, _D_FEAT), lambda i: (i, 0)),
            pl.BlockSpec(w.shape, lambda i: (0, 0)),
        ],
        out_specs=[
            pl.BlockSpec((blk, _D_OUT // 2), lambda i: (i, 0)),
            pl.BlockSpec((blk, _D_OUT // 2), lambda i: (i, 0)),
        ],
        out_shape=[
            jax.ShapeDtypeStruct((_N_NODES, _D_OUT // 2), jnp.int32),
            jax.ShapeDtypeStruct((_N_NODES, _D_OUT // 2), jnp.int32),
        ],
    )(nodes, w)


def _edge_projection(edges_t, w, b2d):
    blk = 16000
    grid = _N_EDGES // blk
    return pl.pallas_call(
        _edge_proj_body,
        grid=(grid,),
        in_specs=[
            pl.BlockSpec((_D_EDGE, blk), lambda i: (0, i)),
            pl.BlockSpec(w.shape, lambda i: (0, 0)),
            pl.BlockSpec((1, _D_OUT), lambda i: (0, 0)),
        ],
        out_specs=pl.BlockSpec((blk, _D_OUT // 2), lambda i: (i, 0)),
        out_shape=jax.ShapeDtypeStruct((_N_EDGES, _D_OUT // 2), jnp.int32),
    )(edges_t, w, b2d)


def _sc_body(pr_hbm, ps_hbm, eb_hbm, recv_hbm, send_hbm, out_hbm,
             idx_r, idx_s,
             r0, s0, e0, o0, r1, s1, e1, o1,
             gsem0, ssem0, gsem1, ssem1):
    wid = lax.axis_index("s") * _NC + lax.axis_index("c")
    base = wid * _EPW
    pltpu.sync_copy(recv_hbm.at[pl.ds(base, _EPW)], idx_r)
    pltpu.sync_copy(send_hbm.at[pl.ds(base, _EPW)], idx_s)

    bufs = ((r0, s0, e0, o0, gsem0, ssem0), (r1, s1, e1, o1, gsem1, ssem1))

    def start_gather(ci, b):
        r, s, e, _o, gs, _ss = bufs[b]
        pltpu.async_copy(pr_hbm.at[idx_r.at[pl.ds(ci * _C, _C)]], r, gs)
        pltpu.async_copy(ps_hbm.at[idx_s.at[pl.ds(ci * _C, _C)]], s, gs)
        pltpu.async_copy(eb_hbm.at[pl.ds(base + ci * _C, _C)], e, gs)

    def wait_gather(b):
        r, s, e, _o, gs, _ss = bufs[b]
        pltpu.make_async_copy(pr_hbm.at[pl.ds(0, _C)], r, gs).wait()
        pltpu.make_async_copy(ps_hbm.at[pl.ds(0, _C)], s, gs).wait()
        pltpu.make_async_copy(eb_hbm.at[pl.ds(0, _C)], e, gs).wait()

    def start_store(ci, b):
        _r, _s, _e, o, _gs, ss = bufs[b]
        pltpu.async_copy(o, out_hbm.at[pl.ds(base + ci * _C, _C)], ss)

    def wait_store(b):
        _r, _s, _e, o, _gs, ss = bufs[b]
        pltpu.make_async_copy(o, out_hbm.at[pl.ds(0, _C)], ss).wait()

    def add_chunk(b):
        r, s, e, o, _gs, _ss = bufs[b]

        def halves(w):
            lo = jax.lax.bitcast_convert_type(w << 16, jnp.float32)
            hi = jax.lax.bitcast_convert_type(w, jnp.float32)
            return lo, hi

        @plsc.parallel_loop(0, _C, 1, unroll=4)
        def _row(i):
            for j in range(_D_OUT // (2 * _L)):
                sl = pl.ds(j * _L, _L)
                ra, rb = halves(r[i, sl])
                sa, sb = halves(s[i, sl])
                ea, ebv = halves(e[i, sl])
                o[i, pl.ds(j * _L, _L)] = ea + ra + sa
                o[i, pl.ds(_D_OUT // 2 + j * _L, _L)] = ebv + rb + sb

    start_gather(0, 0)
    start_gather(1, 1)

    def pair(g, carry):
        for b in (0, 1):
            ci = 2 * g + b
            wait_gather(b)

            @pl.when(g > 0)
            def _():
                wait_store(b)

            add_chunk(b)

            @pl.when(ci + 2 < _NCHUNK)
            def _():
                start_gather(ci + 2, b)

            start_store(ci, b)
        return carry

    lax.fori_loop(0, _NCHUNK // 2, pair, 0)

    wait_gather(0)
    wait_store(0)
    add_chunk(0)
    start_store(_NCHUNK - 1, 0)
    wait_store(0)
    wait_store(1)


_sc_gather_add = functools.partial(
    pl.kernel,
    out_type=jax.ShapeDtypeStruct((_N_EDGES, _D_OUT), jnp.float32),
    mesh=plsc.VectorSubcoreMesh(core_axis_name="c", subcore_axis_name="s"),
    scratch_types=[
        pltpu.VMEM((_EPW,), jnp.int32),
        pltpu.VMEM((_EPW,), jnp.int32),
        pltpu.VMEM((_C, _D_OUT // 2), jnp.int32),
        pltpu.VMEM((_C, _D_OUT // 2), jnp.int32),
        pltpu.VMEM((_C, _D_OUT // 2), jnp.int32),
        pltpu.VMEM((_C, _D_OUT), jnp.float32),
        pltpu.VMEM((_C, _D_OUT // 2), jnp.int32),
        pltpu.VMEM((_C, _D_OUT // 2), jnp.int32),
        pltpu.VMEM((_C, _D_OUT // 2), jnp.int32),
        pltpu.VMEM((_C, _D_OUT), jnp.float32),
        pltpu.SemaphoreType.DMA,
        pltpu.SemaphoreType.DMA,
        pltpu.SemaphoreType.DMA,
        pltpu.SemaphoreType.DMA,
    ],
)(_sc_body)


def kernel(edges, nodes, W, b, senders, receivers):
    pr, ps = _node_projections(nodes, W)
    eb = _edge_projection(edges.T, W, b.reshape(1, _D_OUT))
    return _sc_gather_add(pr, ps, eb, receivers, senders)


# speedup vs baseline: 4.4401x; 1.0024x over previous
import functools

import jax
import jax.numpy as jnp
from jax import lax
from jax.experimental import pallas as pl
from jax.experimental.pallas import tpu as pltpu
from jax.experimental.pallas import tpu_sc as plsc

_N_NODES = 10000
_N_EDGES = 160000
_D_FEAT = 256
_D_EDGE = 16
_D_OUT = 256

_NC = 2
_NS = 16
_L = 16
_NW = _NC * _NS
_EPW = _N_EDGES // _NW
_C = 40
_NCHUNK = _EPW // _C


def _pack_pairs(x):
    u = jax.lax.bitcast_convert_type(x.astype(jnp.bfloat16), jnp.uint16)
    lo = u[:, :_D_OUT // 2].astype(jnp.uint32)
    hi = u[:, _D_OUT // 2:].astype(jnp.uint32)
    return jax.lax.bitcast_convert_type(lo | (hi << 16), jnp.int32)


def _node_proj_body(nodes_ref, w_ref, pr_ref, ps_ref):
    x = nodes_ref[...]
    w_r = w_ref[_D_EDGE:_D_EDGE + _D_FEAT, :]
    w_s = w_ref[_D_EDGE + _D_FEAT:, :]
    pr_ref[...] = _pack_pairs(
        jnp.dot(x, w_r, preferred_element_type=jnp.float32))
    ps_ref[...] = _pack_pairs(
        jnp.dot(x, w_s, preferred_element_type=jnp.float32))


def _edge_proj_body(edges_t_ref, w_ref, b_ref, out_ref):
    w_e = w_ref[:_D_EDGE, :]
    prod = jax.lax.dot_general(
        edges_t_ref[...], w_e, (((0,), (0,)), ((), ())),
        preferred_element_type=jnp.float32)
    out_ref[...] = _pack_pairs(prod + b_ref[...])


def _node_projections(nodes, w):
    blk = 1000
    grid = _N_NODES // blk
    return pl.pallas_call(
        _node_proj_body,
        grid=(grid,),
        in_specs=[
            pl.BlockSpec((blk, _D_FEAT), lambda i: (i, 0)),
            pl.BlockSpec(w.shape, lambda i: (0, 0)),
        ],
        out_specs=[
            pl.BlockSpec((blk, _D_OUT // 2), lambda i: (i, 0)),
            pl.BlockSpec((blk, _D_OUT // 2), lambda i: (i, 0)),
        ],
        out_shape=[
            jax.ShapeDtypeStruct((_N_NODES, _D_OUT // 2), jnp.int32),
            jax.ShapeDtypeStruct((_N_NODES, _D_OUT // 2), jnp.int32),
        ],
    )(nodes, w)


def _edge_projection(edges_t, w, b2d):
    blk = 16000
    grid = _N_EDGES // blk
    return pl.pallas_call(
        _edge_proj_body,
        grid=(grid,),
        in_specs=[
            pl.BlockSpec((_D_EDGE, blk), lambda i: (0, i)),
            pl.BlockSpec(w.shape, lambda i: (0, 0)),
            pl.BlockSpec((1, _D_OUT), lambda i: (0, 0)),
        ],
        out_specs=pl.BlockSpec((blk, _D_OUT // 2), lambda i: (i, 0)),
        out_shape=jax.ShapeDtypeStruct((_N_EDGES, _D_OUT // 2), jnp.int32),
    )(edges_t, w, b2d)


def _sc_body(pr_hbm, ps_hbm, eb_hbm, recv_hbm, send_hbm, out_hbm,
             idx_r, idx_s,
             r0, s0, e0, o0, r1, s1, e1, o1,
             gsem0, ssem0, gsem1, ssem1):
    wid = lax.axis_index("s") * _NC + lax.axis_index("c")
    base = wid * _EPW
    pltpu.sync_copy(recv_hbm.at[pl.ds(base, _EPW)], idx_r)
    pltpu.sync_copy(send_hbm.at[pl.ds(base, _EPW)], idx_s)

    bufs = ((r0, s0, e0, o0, gsem0, ssem0), (r1, s1, e1, o1, gsem1, ssem1))

    def start_gather(ci, b):
        r, s, e, _o, gs, _ss = bufs[b]
        pltpu.async_copy(pr_hbm.at[idx_r.at[pl.ds(ci * _C, _C)]], r, gs)
        pltpu.async_copy(ps_hbm.at[idx_s.at[pl.ds(ci * _C, _C)]], s, gs)
        pltpu.async_copy(eb_hbm.at[pl.ds(base + ci * _C, _C)], e, gs)

    def wait_gather(b):
        r, s, e, _o, gs, _ss = bufs[b]
        pltpu.make_async_copy(pr_hbm.at[pl.ds(0, _C)], r, gs).wait()
        pltpu.make_async_copy(ps_hbm.at[pl.ds(0, _C)], s, gs).wait()
        pltpu.make_async_copy(eb_hbm.at[pl.ds(0, _C)], e, gs).wait()

    def start_store(ci, b):
        _r, _s, _e, o, _gs, ss = bufs[b]
        pltpu.async_copy(o, out_hbm.at[pl.ds(base + ci * _C, _C)], ss)

    def wait_store(b):
        _r, _s, _e, o, _gs, ss = bufs[b]
        pltpu.make_async_copy(o, out_hbm.at[pl.ds(0, _C)], ss).wait()

    def add_chunk(b):
        r, s, e, o, _gs, _ss = bufs[b]

        def halves(w):
            lo = jax.lax.bitcast_convert_type(w << 16, jnp.float32)
            hi = jax.lax.bitcast_convert_type(w, jnp.float32)
            return lo, hi

        @plsc.parallel_loop(0, _C, 1, unroll=8)
        def _row(i):
            for j in range(_D_OUT // (2 * _L)):
                sl = pl.ds(j * _L, _L)
                ra, rb = halves(r[i, sl])
                sa, sb = halves(s[i, sl])
                ea, ebv = halves(e[i, sl])
                o[i, pl.ds(j * _L, _L)] = ea + ra + sa
                o[i, pl.ds(_D_OUT // 2 + j * _L, _L)] = ebv + rb + sb

    start_gather(0, 0)
    start_gather(1, 1)

    def pair(g, carry):
        for b in (0, 1):
            ci = 2 * g + b
            wait_gather(b)

            @pl.when(g > 0)
            def _():
                wait_store(b)

            add_chunk(b)

            @pl.when(ci + 2 < _NCHUNK)
            def _():
                start_gather(ci + 2, b)

            start_store(ci, b)
        return carry

    lax.fori_loop(0, _NCHUNK // 2, pair, 0)

    wait_gather(0)
    wait_store(0)
    add_chunk(0)
    start_store(_NCHUNK - 1, 0)
    wait_store(0)
    wait_store(1)


_sc_gather_add = functools.partial(
    pl.kernel,
    out_type=jax.ShapeDtypeStruct((_N_EDGES, _D_OUT), jnp.float32),
    mesh=plsc.VectorSubcoreMesh(core_axis_name="c", subcore_axis_name="s"),
    scratch_types=[
        pltpu.VMEM((_EPW,), jnp.int32),
        pltpu.VMEM((_EPW,), jnp.int32),
        pltpu.VMEM((_C, _D_OUT // 2), jnp.int32),
        pltpu.VMEM((_C, _D_OUT // 2), jnp.int32),
        pltpu.VMEM((_C, _D_OUT // 2), jnp.int32),
        pltpu.VMEM((_C, _D_OUT), jnp.float32),
        pltpu.VMEM((_C, _D_OUT // 2), jnp.int32),
        pltpu.VMEM((_C, _D_OUT // 2), jnp.int32),
        pltpu.VMEM((_C, _D_OUT // 2), jnp.int32),
        pltpu.VMEM((_C, _D_OUT), jnp.float32),
        pltpu.SemaphoreType.DMA,
        pltpu.SemaphoreType.DMA,
        pltpu.SemaphoreType.DMA,
        pltpu.SemaphoreType.DMA,
    ],
)(_sc_body)


def kernel(edges, nodes, W, b, senders, receivers):
    pr, ps = _node_projections(nodes, W)
    eb = _edge_projection(edges.T, W, b.reshape(1, _D_OUT))
    return _sc_gather_add(pr, ps, eb, receivers, senders)
